# Initial kernel scaffold; baseline (speedup 1.0000x reference)
import jax
import jax.numpy as jnp
from jax.experimental import pallas as pl


def kernel(users, items, user_bias, item_bias):
    raise NotImplementedError("write your pallas kernel here")


# speedup vs baseline: 1.0580x; 1.0580x over previous
import jax
import jax.numpy as jnp
from jax import lax
from jax.experimental import pallas as pl
from jax.experimental.pallas import tpu as pltpu
from jax.experimental.pallas import tpu_sc as plsc

B = 16384
_info = plsc.get_sparse_core_info()
NC, NS, L = _info.num_cores, _info.num_subcores, _info.num_lanes
NW = NC * NS
BPW = B // NW


def _body(users_hbm, items_hbm, ub_hbm, ib_hbm, out_hbm,
          idx_u, idx_i, u_v, i_v, sem_u, sem_i):
    wid = lax.axis_index("s") * NC + lax.axis_index("c")
    base = wid * BPW
    pltpu.sync_copy(users_hbm.at[pl.ds(base, BPW)], idx_u)
    pltpu.sync_copy(items_hbm.at[pl.ds(base, BPW)], idx_i)
    cu = pltpu.async_copy(ub_hbm.at[idx_u], u_v, sem_u)
    ci = pltpu.async_copy(ib_hbm.at[idx_i], i_v, sem_i)
    cu.wait()
    ci.wait()
    for j in range(BPW // L):
        sl = pl.ds(j * L, L)
        u_v[sl] = u_v[sl] + i_v[sl]
    pltpu.sync_copy(u_v, out_hbm.at[pl.ds(base, BPW)])


def kernel(users, items, user_bias, item_bias):
    ub = user_bias.reshape(-1)
    ib = item_bias.reshape(-1)
    run = pl.kernel(
        _body,
        out_type=jax.ShapeDtypeStruct((B,), jnp.float32),
        mesh=plsc.VectorSubcoreMesh(core_axis_name="c", subcore_axis_name="s"),
        scratch_types=[
            pltpu.VMEM((BPW,), jnp.int32),
            pltpu.VMEM((BPW,), jnp.int32),
            pltpu.VMEM((BPW,), jnp.float32),
            pltpu.VMEM((BPW,), jnp.float32),
            pltpu.SemaphoreType.DMA,
            pltpu.SemaphoreType.DMA,
        ],
    )
    return run(users, items, ub, ib)
